# baseline (speedup 1.0000x reference)
import jax
import jax.numpy as jnp
from jax.experimental import pallas as pl
from jax.experimental.pallas import tpu as pltpu

B, N = 1, 1024
DIM_IN, DIM_OUT = 64, 128
KS, STRIDE, RADIUS, SIGMA, NN, NA = 24, 2, 0.4, 0.2, 32, 12
P = N // STRIDE
CK = DIM_IN * KS
PA = P * NA


def _conv_norm_kernel(w_ref, x_ref, o_ref):
    acc = jnp.dot(w_ref[...], x_ref[...], preferred_element_type=jnp.float32)
    mu = jnp.mean(acc, axis=1, keepdims=True)
    var = jnp.mean(acc * acc, axis=1, keepdims=True) - mu * mu
    y = (acc - mu) * jax.lax.rsqrt(var + 1e-5)
    o_ref[...] = jnp.maximum(y, 0.0)


def kernel(xyz, feats, anchors, W, kernels):
    b, c, n, na = feats.shape
    sample_idx = jnp.arange(0, n, STRIDE)
    x_t = jnp.transpose(xyz, (0, 2, 1))
    nx = x_t[:, ::STRIDE, :]
    new_xyz = jnp.transpose(nx, (0, 2, 1))

    d2 = jnp.sum((nx[:, :, None, :] - x_t[:, None, :, :]) ** 2, axis=-1)
    _, inter_idx = jax.lax.top_k(-d2, NN)
    grouped = jnp.take_along_axis(x_t[:, None, :, :], inter_idx[..., None], axis=2)
    rel = grouped - nx[:, :, None, :]

    rk = jnp.einsum('aij,kj->aki', anchors, kernels)
    diff = rel[:, :, :, None, None, :] - rk[None, None, None, :, :, :]
    dist = jnp.sqrt(jnp.sum(diff * diff, axis=-1) + 1e-12)
    inter_w = jnp.maximum(1.0 - dist / SIGMA, 0.0)

    f_t = jnp.transpose(feats, (0, 2, 1, 3))
    gf = jnp.take_along_axis(f_t[:, None, :, :, :], inter_idx[:, :, :, None, None], axis=2)
    new_f = jnp.einsum('bpnca,bpnak->bckpa', gf, inter_w)
    conv_in = jnp.reshape(new_f, (CK, PA)).astype(jnp.bfloat16)

    out = pl.pallas_call(
        _conv_norm_kernel,
        out_shape=jax.ShapeDtypeStruct((DIM_OUT, PA), jnp.float32),
    )(W.astype(jnp.bfloat16), conv_in)

    feat = jnp.reshape(out, (B, DIM_OUT, P, NA))
    return inter_idx, inter_w, sample_idx, new_xyz, feat, anchors
